# baseline (speedup 1.0000x reference)
import jax
import jax.numpy as jnp
from jax import lax
from jax.experimental import pallas as pl
from jax.experimental.pallas import tpu as pltpu
from jax.experimental.pallas import tpu_sc as plsc

_NUM_DIM = 10_000_000
_NUM_ACTIVE = 5_000_000
_NC = 2
_NS = 16
_NW = _NC * _NS
_CHUNK = 8000
_NFULL = 19
_BLEN = 4240
_TLEN = 160
_A_TOTAL = _NFULL * _NW * _CHUNK
_B_TOTAL = _A_TOTAL + _NW * _BLEN
_LENS = [_CHUNK] * _NFULL + [_BLEN, _TLEN]
_GMAX = len(_LENS)
_LANES = 16
_UNROLL = 4

_mesh = plsc.VectorSubcoreMesh(core_axis_name="c", subcore_axis_name="s")


@pl.kernel(
    out_type=jax.ShapeDtypeStruct((_NW, _LANES), jnp.float32),
    mesh=_mesh,
    scratch_types=[
        pltpu.VMEM((_CHUNK,), jnp.int32),
        pltpu.VMEM((_CHUNK,), jnp.int32),
        pltpu.VMEM((_CHUNK,), jnp.int32),
        pltpu.VMEM((_CHUNK,), jnp.float32),
        pltpu.VMEM((_CHUNK,), jnp.float32),
        pltpu.VMEM((_CHUNK,), jnp.float32),
        pltpu.VMEM((_LANES,), jnp.float32),
        pltpu.SemaphoreType.DMA,
        pltpu.SemaphoreType.DMA,
        pltpu.SemaphoreType.DMA,
        pltpu.SemaphoreType.DMA,
        pltpu.SemaphoreType.DMA,
        pltpu.SemaphoreType.DMA,
    ],
)
def _gather_sq_partials(idx_hbm, table_hbm, x0_hbm, out_hbm,
                        idx0, idx1, idx2, rows0, rows1, rows2, stage,
                        si0, si1, si2, sg0, sg1, sg2):
    wid = lax.axis_index("s") * _NC + lax.axis_index("c")
    pltpu.sync_copy(x0_hbm, stage.at[pl.ds(0, 1)])
    x0 = jnp.full((_LANES,), stage[...][0], dtype=jnp.float32)

    idx_b = (idx0, idx1, idx2)
    rows_b = (rows0, rows1, rows2)
    si = (si0, si1, si2)
    sg = (sg0, sg1, sg2)
    _NB = 3

    def offset(g):
        if g < _NFULL:
            return (wid + g * _NW) * _CHUNK
        if g == _NFULL:
            return _A_TOTAL + wid * _BLEN
        return _B_TOTAL + jnp.minimum(wid, 1) * _TLEN

    def start_idx(g):
        b = g % _NB
        return pltpu.async_copy(
            idx_hbm.at[pl.ds(offset(g), _LENS[g])],
            idx_b[b].at[pl.ds(0, _LENS[g])], si[b])

    def start_gather(g):
        b = g % _NB
        return pltpu.async_copy(
            table_hbm.at[idx_b[b].at[pl.ds(0, _LENS[g])]],
            rows_b[b].at[pl.ds(0, _LENS[g])], sg[b])

    def reduce_chunk(rows, n):
        z = jnp.zeros((_LANES,), jnp.float32)
        k4 = n // (_LANES * _UNROLL)

        def inner(i, accs):
            base = i * (_LANES * _UNROLL)
            out = []
            for u in range(_UNROLL):
                v = rows[pl.ds(base + u * _LANES, _LANES)]
                d = v - x0
                out.append(accs[u] + d * d)
            return tuple(out)

        accs = lax.fori_loop(0, k4, inner, (z,) * _UNROLL)
        csum = (accs[0] + accs[1]) + (accs[2] + accs[3])
        for j in range(k4 * _UNROLL, n // _LANES):
            v = rows[pl.ds(j * _LANES, _LANES)]
            d = v - x0
            csum = csum + d * d
        return csum

    pend_i = [start_idx(g) for g in range(min(_NB, _GMAX))]
    pend_g = {}
    for g in range(min(2, _GMAX)):
        pend_i[g].wait()
        pend_g[g] = start_gather(g)

    acc = jnp.zeros((_LANES,), jnp.float32)
    for g in range(_GMAX):
        if g + 2 < _GMAX:
            pend_i[(g + 2) % _NB].wait()
            pend_g[g + 2] = start_gather(g + 2)
        pend_g.pop(g).wait()
        if g + _NB < _GMAX:
            pend_i[g % _NB] = start_idx(g + _NB)
        csum = reduce_chunk(rows_b[g % _NB], _LENS[g])
        if g == _GMAX - 1:
            csum = jnp.where(wid < 2, csum, jnp.zeros_like(csum))
        acc = acc + csum

    stage[...] = acc
    pltpu.sync_copy(stage, out_hbm.at[wid])


def kernel(parameters_, active_idx, x_0):
    x0_arr = jnp.reshape(x_0.astype(jnp.float32), (1,))
    partials = _gather_sq_partials(active_idx, parameters_, x0_arr)
    return -jnp.sum(partials)


# speedup vs baseline: 1.0020x; 1.0020x over previous
import jax
import jax.numpy as jnp
from jax import lax
from jax.experimental import pallas as pl
from jax.experimental.pallas import tpu as pltpu
from jax.experimental.pallas import tpu_sc as plsc

_NUM_DIM = 10_000_000
_NUM_ACTIVE = 5_000_000
_NC = 2
_NS = 16
_NW = _NC * _NS
_CHUNK = 8000
_NFULL = 19
_BLEN = 4240
_TLEN = 160
_A_TOTAL = _NFULL * _NW * _CHUNK
_B_TOTAL = _A_TOTAL + _NW * _BLEN
_LENS = [_CHUNK] * _NFULL + [_BLEN, _TLEN]
_GMAX = len(_LENS)
_LANES = 16
_UNROLL = 4

_mesh = plsc.VectorSubcoreMesh(core_axis_name="c", subcore_axis_name="s")


@pl.kernel(
    out_type=jax.ShapeDtypeStruct((_NW, _LANES), jnp.float32),
    mesh=_mesh,
    scratch_types=[
        pltpu.VMEM((_CHUNK,), jnp.int32),
        pltpu.VMEM((_CHUNK,), jnp.int32),
        pltpu.VMEM((_CHUNK,), jnp.int32),
        pltpu.VMEM((_CHUNK,), jnp.float32),
        pltpu.VMEM((_CHUNK,), jnp.float32),
        pltpu.VMEM((_CHUNK,), jnp.float32),
        pltpu.VMEM((_LANES,), jnp.float32),
        pltpu.SemaphoreType.DMA,
        pltpu.SemaphoreType.DMA,
        pltpu.SemaphoreType.DMA,
        pltpu.SemaphoreType.DMA,
        pltpu.SemaphoreType.DMA,
        pltpu.SemaphoreType.DMA,
    ],
)
def _gather_sq_partials(idx_hbm, table_hbm, x0_hbm, out_hbm,
                        idx0, idx1, idx2, rows0, rows1, rows2, stage,
                        si0, si1, si2, sg0, sg1, sg2):
    wid = lax.axis_index("s") * _NC + lax.axis_index("c")
    idx_b = (idx0, idx1, idx2)
    rows_b = (rows0, rows1, rows2)
    si = (si0, si1, si2)
    sg = (sg0, sg1, sg2)
    _NB = 3

    def offset(g):
        if g < _NFULL:
            return (wid + g * _NW) * _CHUNK
        if g == _NFULL:
            return _A_TOTAL + wid * _BLEN
        return _B_TOTAL + jnp.minimum(wid, 1) * _TLEN

    def start_idx(g):
        b = g % _NB
        return pltpu.async_copy(
            idx_hbm.at[pl.ds(offset(g), _LENS[g])],
            idx_b[b].at[pl.ds(0, _LENS[g])], si[b])

    def start_gather(g):
        b = g % _NB
        return pltpu.async_copy(
            table_hbm.at[idx_b[b].at[pl.ds(0, _LENS[g])]],
            rows_b[b].at[pl.ds(0, _LENS[g])], sg[b])

    def reduce_chunk(rows, n):
        z = jnp.zeros((_LANES,), jnp.float32)
        k4 = n // (_LANES * _UNROLL)

        def inner(i, accs):
            base = i * (_LANES * _UNROLL)
            out = []
            for u in range(_UNROLL):
                v = rows[pl.ds(base + u * _LANES, _LANES)]
                d = v - x0
                out.append(accs[u] + d * d)
            return tuple(out)

        accs = lax.fori_loop(0, k4, inner, (z,) * _UNROLL)
        csum = (accs[0] + accs[1]) + (accs[2] + accs[3])
        for j in range(k4 * _UNROLL, n // _LANES):
            v = rows[pl.ds(j * _LANES, _LANES)]
            d = v - x0
            csum = csum + d * d
        return csum

    pend_i = [start_idx(g) for g in range(min(_NB, _GMAX))]
    pltpu.sync_copy(x0_hbm, stage.at[pl.ds(0, 1)])
    x0 = jnp.full((_LANES,), stage[...][0], dtype=jnp.float32)
    pend_g = {}
    for g in range(min(2, _GMAX)):
        pend_i[g].wait()
        pend_g[g] = start_gather(g)

    acc = jnp.zeros((_LANES,), jnp.float32)
    for g in range(_GMAX):
        if g + 2 < _GMAX:
            pend_i[(g + 2) % _NB].wait()
            pend_g[g + 2] = start_gather(g + 2)
        pend_g.pop(g).wait()
        if g + _NB < _GMAX:
            pend_i[g % _NB] = start_idx(g + _NB)
        csum = reduce_chunk(rows_b[g % _NB], _LENS[g])
        if g == _GMAX - 1:
            csum = jnp.where(wid < 2, csum, jnp.zeros_like(csum))
        acc = acc + csum

    stage[...] = acc
    pltpu.sync_copy(stage, out_hbm.at[wid])


def kernel(parameters_, active_idx, x_0):
    x0_arr = jnp.reshape(x_0.astype(jnp.float32), (1,))
    partials = _gather_sq_partials(active_idx, parameters_, x0_arr)
    return -jnp.sum(partials)


# speedup vs baseline: 1.0033x; 1.0013x over previous
import jax
import jax.numpy as jnp
from jax import lax
from jax.experimental import pallas as pl
from jax.experimental.pallas import tpu as pltpu
from jax.experimental.pallas import tpu_sc as plsc

_NUM_DIM = 10_000_000
_NUM_ACTIVE = 5_000_000
_NC = 2
_NS = 16
_NW = _NC * _NS
_CHUNK = 8000
_NFULL = 19
_BLEN = 4240
_TLEN = 160
_A_TOTAL = _NFULL * _NW * _CHUNK
_B_TOTAL = _A_TOTAL + _NW * _BLEN
_LENS = [_CHUNK] * _NFULL + [_BLEN, _TLEN]
_GMAX = len(_LENS)
_LANES = 16
_UNROLL = 4

_mesh = plsc.VectorSubcoreMesh(core_axis_name="c", subcore_axis_name="s")


@pl.kernel(
    out_type=jax.ShapeDtypeStruct((_NW, _LANES), jnp.float32),
    mesh=_mesh,
    scratch_types=[
        pltpu.VMEM((_CHUNK,), jnp.int32),
        pltpu.VMEM((_CHUNK,), jnp.int32),
        pltpu.VMEM((_CHUNK,), jnp.int32),
        pltpu.VMEM((_CHUNK,), jnp.float32),
        pltpu.VMEM((_CHUNK,), jnp.float32),
        pltpu.VMEM((_CHUNK,), jnp.float32),
        pltpu.VMEM((_LANES,), jnp.float32),
        pltpu.SemaphoreType.DMA,
        pltpu.SemaphoreType.DMA,
        pltpu.SemaphoreType.DMA,
        pltpu.SemaphoreType.DMA,
        pltpu.SemaphoreType.DMA,
        pltpu.SemaphoreType.DMA,
        pltpu.SemaphoreType.DMA,
    ],
)
def _gather_sq_partials(idx_hbm, table_hbm, x0_hbm, out_hbm,
                        idx0, idx1, idx2, rows0, rows1, rows2, stage,
                        si0, si1, si2, sg0, sg1, sg2, sx0):
    wid = lax.axis_index("s") * _NC + lax.axis_index("c")
    idx_b = (idx0, idx1, idx2)
    rows_b = (rows0, rows1, rows2)
    si = (si0, si1, si2)
    sg = (sg0, sg1, sg2)
    _NB = 3

    def offset(g):
        if g < _NFULL:
            return (wid + g * _NW) * _CHUNK
        if g == _NFULL:
            return _A_TOTAL + wid * _BLEN
        return _B_TOTAL + jnp.minimum(wid, 1) * _TLEN

    def start_idx(g):
        b = g % _NB
        return pltpu.async_copy(
            idx_hbm.at[pl.ds(offset(g), _LENS[g])],
            idx_b[b].at[pl.ds(0, _LENS[g])], si[b])

    def start_gather(g):
        b = g % _NB
        return pltpu.async_copy(
            table_hbm.at[idx_b[b].at[pl.ds(0, _LENS[g])]],
            rows_b[b].at[pl.ds(0, _LENS[g])], sg[b])

    def reduce_chunk(rows, n):
        z = jnp.zeros((_LANES,), jnp.float32)
        k4 = n // (_LANES * _UNROLL)

        def inner(i, accs):
            base = i * (_LANES * _UNROLL)
            out = []
            for u in range(_UNROLL):
                v = rows[pl.ds(base + u * _LANES, _LANES)]
                d = v - x0
                out.append(accs[u] + d * d)
            return tuple(out)

        accs = lax.fori_loop(0, k4, inner, (z,) * _UNROLL)
        csum = (accs[0] + accs[1]) + (accs[2] + accs[3])
        for j in range(k4 * _UNROLL, n // _LANES):
            v = rows[pl.ds(j * _LANES, _LANES)]
            d = v - x0
            csum = csum + d * d
        return csum

    pend_x0 = pltpu.async_copy(x0_hbm, stage.at[pl.ds(0, 1)], sx0)
    pend_i = [start_idx(g) for g in range(min(_NB, _GMAX))]
    pend_x0.wait()
    x0 = jnp.full((_LANES,), stage[...][0], dtype=jnp.float32)
    pend_g = {}
    for g in range(min(2, _GMAX)):
        pend_i[g].wait()
        pend_g[g] = start_gather(g)

    acc = jnp.zeros((_LANES,), jnp.float32)
    for g in range(_GMAX):
        if g + 2 < _GMAX:
            pend_i[(g + 2) % _NB].wait()
            pend_g[g + 2] = start_gather(g + 2)
        pend_g.pop(g).wait()
        if g + _NB < _GMAX:
            pend_i[g % _NB] = start_idx(g + _NB)
        csum = reduce_chunk(rows_b[g % _NB], _LENS[g])
        if g == _GMAX - 1:
            csum = jnp.where(wid < 2, csum, jnp.zeros_like(csum))
        acc = acc + csum

    stage[...] = acc
    pltpu.sync_copy(stage, out_hbm.at[wid])


def kernel(parameters_, active_idx, x_0):
    x0_arr = jnp.reshape(x_0.astype(jnp.float32), (1,))
    partials = _gather_sq_partials(active_idx, parameters_, x0_arr)
    return -jnp.sum(partials)
